# baseline (speedup 1.0000x reference)
import jax
import jax.numpy as jnp
import numpy as np
from jax.experimental import pallas as pl
from jax.experimental.pallas import tpu as pltpu

_N = 62
_D = 16
_B = 1024
_H = 400
_C = 2
_BB = 256
_PPAD = 2048


def _build_w(p_row):
    rows = [p_row[:, r * (r + 1) // 2: r * (r + 1) // 2 + _N] for r in range(_N)]
    p2 = jnp.concatenate(rows, axis=0)
    i2 = jax.lax.broadcasted_iota(jnp.int32, (_N, _N), 0)
    j2 = jax.lax.broadcasted_iota(jnp.int32, (_N, _N), 1)
    return jnp.where(i2 >= j2, p2, p2.T)


def _fused_body(xt_ref, p_ref, lwa_ref, fw_ref, fb_ref, o_ref, a2_ref):
    @pl.when(pl.program_id(0) == 0)
    def _compute_a2():
        w = _build_w(p_ref[...])
        aw = jnp.abs(w)
        deg_c = jnp.sum(aw, axis=1, keepdims=True)
        deg_r = jnp.sum(aw, axis=0, keepdims=True)
        dinv_c = jnp.where(deg_c > 0, jax.lax.rsqrt(deg_c), 0.0)
        dinv_r = jnp.where(deg_r > 0, jax.lax.rsqrt(deg_r), 0.0)
        a = w * dinv_c * dinv_r
        a2_ref[...] = jnp.dot(a, a, preferred_element_type=jnp.float32)

    a2 = a2_ref[...]
    cb = _BB // 4
    for c in range(4):
        xbd = xt_ref[c * cb * _D:(c + 1) * cb * _D, :]
        y = jnp.dot(xbd, a2, preferred_element_type=jnp.float32)
        y2 = y.reshape(cb, _D, _N).transpose(0, 2, 1).reshape(cb * _N, _D)
        ones = jnp.ones((cb * _N, 1), jnp.float32)
        ya = jnp.concatenate([y2, ones], axis=1)
        h = jnp.maximum(
            jnp.dot(ya, lwa_ref[...], preferred_element_type=jnp.float32), 0.0
        )
        o1 = jnp.dot(h, fw_ref[...], preferred_element_type=jnp.float32)
        o_ref[c * cb:(c + 1) * cb, :] = (
            jnp.sum(o1.reshape(cb, _N, _C), axis=1) + fb_ref[...]
        )


def kernel(X, X2, padding_masks, edge_weight_param, base_edge_index, lin_w, lin_b, fc_w, fc_b):
    nt = _N * (_N + 1) // 2
    p_row = jnp.pad(edge_weight_param, (0, _PPAD - nt)).reshape(1, _PPAD)
    xt = X.transpose(0, 2, 1).reshape(_B * _D, _N)
    lwa = jnp.concatenate([lin_w, lin_b.reshape(1, _H)], axis=0)

    grid = (_B // _BB,)
    out = pl.pallas_call(
        _fused_body,
        grid=grid,
        in_specs=[
            pl.BlockSpec((_BB * _D, _N), lambda i: (i, 0)),
            pl.BlockSpec((1, _PPAD), lambda i: (0, 0)),
            pl.BlockSpec((_D + 1, _H), lambda i: (0, 0)),
            pl.BlockSpec((_H, _C), lambda i: (0, 0)),
            pl.BlockSpec((1, _C), lambda i: (0, 0)),
        ],
        out_specs=pl.BlockSpec((_BB, _C), lambda i: (i, 0)),
        out_shape=jax.ShapeDtypeStruct((_B, _C), jnp.float32),
        scratch_shapes=[pltpu.VMEM((_N, _N), jnp.float32)],
    )(xt, p_row, lwa, fc_w, fc_b.reshape(1, _C))
    return out


# speedup vs baseline: 1.0385x; 1.0385x over previous
import jax
import jax.numpy as jnp
import numpy as np
from jax.experimental import pallas as pl
from jax.experimental.pallas import tpu as pltpu

_N = 62
_D = 16
_B = 1024
_H = 400
_C = 2
_BB = 512
_PPAD = 2048


def _build_w(p_row):
    rows = [p_row[:, r * (r + 1) // 2: r * (r + 1) // 2 + _N] for r in range(_N)]
    p2 = jnp.concatenate(rows, axis=0)
    i2 = jax.lax.broadcasted_iota(jnp.int32, (_N, _N), 0)
    j2 = jax.lax.broadcasted_iota(jnp.int32, (_N, _N), 1)
    return jnp.where(i2 >= j2, p2, p2.T)


def _fused_body(xt_ref, p_ref, lwa_ref, fw_ref, fb_ref, o_ref, a2_ref):
    @pl.when(pl.program_id(0) == 0)
    def _compute_a2():
        w = _build_w(p_ref[...])
        aw = jnp.abs(w)
        deg_c = jnp.sum(aw, axis=1, keepdims=True)
        deg_r = jnp.sum(aw, axis=0, keepdims=True)
        dinv_c = jnp.where(deg_c > 0, jax.lax.rsqrt(deg_c), 0.0)
        dinv_r = jnp.where(deg_r > 0, jax.lax.rsqrt(deg_r), 0.0)
        a = w * dinv_c * dinv_r
        a2_ref[...] = jnp.dot(a, a, preferred_element_type=jnp.float32)

    a2 = a2_ref[...]
    cb = _BB // 2
    for c in range(2):
        xbd = xt_ref[c * cb * _D:(c + 1) * cb * _D, :]
        y = jnp.dot(xbd, a2, preferred_element_type=jnp.float32)
        y2 = y.reshape(cb, _D, _N).transpose(0, 2, 1).reshape(cb * _N, _D)
        ones = jnp.ones((cb * _N, 1), jnp.float32)
        ya = jnp.concatenate([y2, ones], axis=1)
        h = jnp.maximum(
            jnp.dot(ya, lwa_ref[...], preferred_element_type=jnp.float32), 0.0
        )
        o1 = jnp.dot(h, fw_ref[...], preferred_element_type=jnp.float32)
        o_ref[c * cb:(c + 1) * cb, :] = (
            jnp.sum(o1.reshape(cb, _N, _C), axis=1) + fb_ref[...]
        )


def kernel(X, X2, padding_masks, edge_weight_param, base_edge_index, lin_w, lin_b, fc_w, fc_b):
    nt = _N * (_N + 1) // 2
    p_row = jnp.pad(edge_weight_param, (0, _PPAD - nt)).reshape(1, _PPAD)
    xt = X.transpose(0, 2, 1).reshape(_B * _D, _N)
    lwa = jnp.concatenate([lin_w, lin_b.reshape(1, _H)], axis=0)

    grid = (_B // _BB,)
    out = pl.pallas_call(
        _fused_body,
        grid=grid,
        in_specs=[
            pl.BlockSpec((_BB * _D, _N), lambda i: (i, 0)),
            pl.BlockSpec((1, _PPAD), lambda i: (0, 0)),
            pl.BlockSpec((_D + 1, _H), lambda i: (0, 0)),
            pl.BlockSpec((_H, _C), lambda i: (0, 0)),
            pl.BlockSpec((1, _C), lambda i: (0, 0)),
        ],
        out_specs=pl.BlockSpec((_BB, _C), lambda i: (i, 0)),
        out_shape=jax.ShapeDtypeStruct((_B, _C), jnp.float32),
        scratch_shapes=[pltpu.VMEM((_N, _N), jnp.float32)],
    )(xt, p_row, lwa, fc_w, fc_b.reshape(1, _C))
    return out
